# baseline (speedup 1.0000x reference)
import functools

import jax
import jax.numpy as jnp
from jax import lax
from jax.experimental import pallas as pl
from jax.experimental.pallas import tpu as pltpu
from jax.experimental.pallas import tpu_sc as plsc

B, C, N, K = 1024, 128, 24, 4
R = B * N
F = 64
NW = 32
BPW = B // NW
CB = 4
CR = CB * N
NCHUNK = BPW // CB
NGRP = CR // 16


def _mm_body(x_ref, w_ref, b_ref, o_ref):
    o_ref[...] = (
        jnp.dot(x_ref[...], w_ref[...], preferred_element_type=jnp.float32)
        + b_ref[...]
    )


def _matmul(x, w, b, blk=2048):
    rows, cin = x.shape
    cout = w.shape[1]
    return pl.pallas_call(
        _mm_body,
        grid=(rows // blk,),
        in_specs=[
            pl.BlockSpec((blk, cin), lambda i: (i, 0)),
            pl.BlockSpec((cin, cout), lambda i: (0, 0)),
            pl.BlockSpec((1, cout), lambda i: (0, 0)),
        ],
        out_specs=pl.BlockSpec((blk, cout), lambda i: (i, 0)),
        out_shape=jax.ShapeDtypeStruct((rows, cout), jnp.float32),
    )(x, w, b)


def _sc_gather_max(pq_flat, gtx, rows=R):
    mesh = plsc.VectorSubcoreMesh(
        core_axis_name="c", subcore_axis_name="s", num_cores=2, num_subcores=16
    )

    nchunk = rows // (NW * CR)

    @functools.partial(
        pl.kernel,
        out_type=jax.ShapeDtypeStruct((rows, F), jnp.float32),
        mesh=mesh,
        compiler_params=pltpu.CompilerParams(needs_layout_passes=False),
        scratch_types=[
            pltpu.VMEM((2 * CR * 2 * F,), jnp.float32),
            pltpu.VMEM((2 * CR, K * 16), jnp.int32),
            pltpu.VMEM((2 * CR, F), jnp.float32),
            pltpu.SemaphoreType.DMA((2,)),
            pltpu.SemaphoreType.DMA((2,)),
        ],
    )
    def run(pq_hbm, gtx_hbm, out_hbm, pq_v, gtx_v, h_v, isem, osem):
        wid = lax.axis_index("s") * 2 + lax.axis_index("c")
        PQB = CR * 2 * F

        def issue_in(c, par):
            ch = wid * nchunk + c
            pltpu.async_copy(
                pq_hbm.at[pl.ds(ch * PQB, PQB)],
                pq_v.at[pl.ds(par * PQB, PQB)],
                isem.at[par],
            )
            pltpu.async_copy(
                gtx_hbm.at[pl.ds(ch * CR, CR), :],
                gtx_v.at[pl.ds(par * CR, CR), :],
                isem.at[par],
            )

        issue_in(0, 0)

        def chunk_body(c, carry):
            par = c % 2
            ch = wid * nchunk + c
            pofs = par * PQB
            rofs = par * CR

            @pl.when(c + 1 < nchunk)
            def _prefetch():
                issue_in(c + 1, 1 - par)

            pltpu.make_async_copy(
                pq_hbm.at[pl.ds(0, PQB)],
                pq_v.at[pl.ds(0, PQB)],
                isem.at[par],
            ).wait()
            pltpu.make_async_copy(
                gtx_hbm.at[pl.ds(0, CR), :],
                gtx_v.at[pl.ds(0, CR), :],
                isem.at[par],
            ).wait()

            @pl.when(c >= 2)
            def _drain_out():
                pltpu.make_async_copy(
                    h_v.at[pl.ds(0, CR), :],
                    out_hbm.at[pl.ds(0, CR), :],
                    osem.at[par],
                ).wait()

            @plsc.parallel_loop(0, CR, step=4)
            def _rows(r0):
                for j in range(4):
                    rr = r0 + j
                    rbase = pofs + rr * (2 * F) + F
                    bv = [
                        gtx_v[rofs + rr, pl.ds(kk * 16, 16)] + pofs
                        for kk in range(K)
                    ]
                    for f0 in range(0, F, 16):
                        m = plsc.load_gather(pq_v, [bv[0] + f0])
                        for kk in range(1, K):
                            m = jnp.maximum(
                                m, plsc.load_gather(pq_v, [bv[kk] + f0])
                            )
                        h = m + pq_v[pl.ds(rbase + f0, 16)]
                        h = jnp.where(h >= 0, h, 0.2 * h)
                        h_v[rofs + rr, pl.ds(f0, 16)] = h
            pltpu.async_copy(
                h_v.at[pl.ds(rofs, CR), :],
                out_hbm.at[pl.ds(ch * CR, CR), :],
                osem.at[par],
            )
            return carry

        lax.fori_loop(0, nchunk, chunk_body, 0)
        for par in range(2):
            pltpu.make_async_copy(
                h_v.at[pl.ds(0, CR), :],
                out_hbm.at[pl.ds(0, CR), :],
                osem.at[par],
            ).wait()

    return run(pq_flat, gtx)


def _sc_assemble(h1a, h1b, h2a, h2b, h3a, h3b):
    mesh = plsc.VectorSubcoreMesh(
        core_axis_name="c", subcore_axis_name="s", num_cores=2, num_subcores=16
    )
    HROWS = R // 2

    @functools.partial(
        pl.kernel,
        out_type=jax.ShapeDtypeStruct((B * 3 * F, N), jnp.float32),
        mesh=mesh,
        compiler_params=pltpu.CompilerParams(needs_layout_passes=False),
        scratch_types=[
            pltpu.VMEM((CR, F), jnp.float32),
            pltpu.VMEM((CR, F + 1), jnp.float32),
            pltpu.VMEM((CB * 3 * F, N), jnp.float32),
        ],
    )
    def run(r1a, r1b, r2a, r2b, r3a, r3b, out_hbm, hst, hp, outc):
        wid = lax.axis_index("s") * 2 + lax.axis_index("c")
        lane = lax.iota(jnp.int32, 16)
        rvv = [g * 16 + lane for g in range(NGRP)]
        cb192 = [(rv // N) * (3 * F) for rv in rvv]
        nvv = [rv % N for rv in rvv]

        def chunk_body(c, carry):
            b0 = wid * BPW + c * CB
            rg = b0 * N
            rl = rg - (wid // (NW // 2)) * HROWS
            for l, (ra, rb) in enumerate(
                ((r1a, r1b), (r2a, r2b), (r3a, r3b))
            ):
                @pl.when(wid < NW // 2)
                def _stage_a():
                    pltpu.sync_copy(ra.at[pl.ds(rl, CR), :], hst)

                @pl.when(wid >= NW // 2)
                def _stage_b():
                    pltpu.sync_copy(rb.at[pl.ds(rl, CR), :], hst)

                def rp_body(a, carry2):
                    r0 = a * 4
                    for j in range(4):
                        rr = r0 + j
                        for f0 in range(0, F, 16):
                            v = hst[rr, pl.ds(f0, 16)]
                            plsc.store_scatter(
                                hp, [jnp.full((16,), rr, jnp.int32),
                                     f0 + lane], v
                            )
                    return carry2

                lax.fori_loop(0, CR // 4, rp_body, 0)

                lF = l * F

                @plsc.parallel_loop(0, F, step=1)
                def _tp(f):
                    fsp = jnp.zeros((16,), jnp.int32) + f
                    for g in range(NGRP):
                        vec = plsc.load_gather(hp, [rvv[g], fsp])
                        plsc.store_scatter(
                            outc, [cb192[g] + (lF + f), nvv[g]], vec
                        )

            pltpu.sync_copy(
                outc, out_hbm.at[pl.ds(b0 * 3 * F, CB * 3 * F), :]
            )
            return carry

        lax.fori_loop(0, NCHUNK, chunk_body, 0)

    return run(h1a, h1b, h2a, h2b, h3a, h3b)


def _combine(W, b, cin):
    Wc = jnp.concatenate([W[:cin], W[cin:] - W[:cin]], axis=1)
    bc = jnp.concatenate([jnp.zeros((F,), jnp.float32), b]).reshape(1, 2 * F)
    return Wc, bc


def kernel(x, idx, W1, b1, W2, b2, W3, b3):
    xt = jnp.transpose(x, (0, 2, 1)).reshape(R, C)
    Wc1, bc1 = _combine(W1, b1, C)
    Wc2, bc2 = _combine(W2, b2, F)
    Wc3, bc3 = _combine(W3, b3, F)
    loc = (jnp.arange(B, dtype=jnp.int32) % CB)[:, None, None] * N + idx
    gtx = (
        loc.reshape(R, K)[:, :, None] * (2 * F)
        + jnp.arange(16, dtype=jnp.int32)
    ).reshape(R, K * 16)

    H = R // 2
    halves_h = []
    for xh, gh in ((xt[:H], gtx[:H]), (xt[H:], gtx[H:])):
        hs = []
        h = xh
        for Wc, bc in ((Wc1, bc1), (Wc2, bc2), (Wc3, bc3)):
            pq = _matmul(h, Wc, bc)
            h = _sc_gather_max(pq.reshape(-1), gh, rows=H)
            hs.append(h)
        halves_h.append(hs)

    out2 = _sc_assemble(
        halves_h[0][0], halves_h[1][0],
        halves_h[0][1], halves_h[1][1],
        halves_h[0][2], halves_h[1][2],
    )
    return out2.reshape(B, 3 * F, N)


# speedup vs baseline: 1.0821x; 1.0821x over previous
import functools

import jax
import jax.numpy as jnp
from jax import lax
from jax.experimental import pallas as pl
from jax.experimental.pallas import tpu as pltpu
from jax.experimental.pallas import tpu_sc as plsc

B, C, N, K = 1024, 128, 24, 4
R = B * N
F = 64
NW = 32
BPW = B // NW
CB = 4
CR = CB * N
NCHUNK = BPW // CB
NGRP = CR // 16


def _mm_body(x_ref, w_ref, b_ref, o_ref):
    o_ref[...] = (
        jnp.dot(x_ref[...], w_ref[...], preferred_element_type=jnp.float32)
        + b_ref[...]
    )


def _matmul(x, w, b, blk=2048):
    rows, cin = x.shape
    cout = w.shape[1]
    return pl.pallas_call(
        _mm_body,
        grid=(rows // blk,),
        in_specs=[
            pl.BlockSpec((blk, cin), lambda i: (i, 0)),
            pl.BlockSpec((cin, cout), lambda i: (0, 0)),
            pl.BlockSpec((1, cout), lambda i: (0, 0)),
        ],
        out_specs=pl.BlockSpec((blk, cout), lambda i: (i, 0)),
        out_shape=jax.ShapeDtypeStruct((rows, cout), jnp.float32),
    )(x, w, b)


def _sc_gather_max(pq_flat, gtx, rows=R):
    mesh = plsc.VectorSubcoreMesh(
        core_axis_name="c", subcore_axis_name="s", num_cores=2, num_subcores=16
    )

    nchunk = rows // (NW * CR)

    @functools.partial(
        pl.kernel,
        out_type=jax.ShapeDtypeStruct((rows, F), jnp.float32),
        mesh=mesh,
        compiler_params=pltpu.CompilerParams(needs_layout_passes=False),
        scratch_types=[
            pltpu.VMEM((2 * CR * 2 * F,), jnp.float32),
            pltpu.VMEM((2 * CR, K * 16), jnp.int32),
            pltpu.VMEM((2 * CR, F), jnp.float32),
            pltpu.SemaphoreType.DMA((2,)),
            pltpu.SemaphoreType.DMA((2,)),
        ],
    )
    def run(pq_hbm, gtx_hbm, out_hbm, pq_v, gtx_v, h_v, isem, osem):
        wid = lax.axis_index("s") * 2 + lax.axis_index("c")
        PQB = CR * 2 * F

        def issue_in(c, par):
            ch = wid * nchunk + c
            pltpu.async_copy(
                pq_hbm.at[pl.ds(ch * PQB, PQB)],
                pq_v.at[pl.ds(par * PQB, PQB)],
                isem.at[par],
            )
            pltpu.async_copy(
                gtx_hbm.at[pl.ds(ch * CR, CR), :],
                gtx_v.at[pl.ds(par * CR, CR), :],
                isem.at[par],
            )

        issue_in(0, 0)

        def chunk_body(c, carry):
            par = c % 2
            ch = wid * nchunk + c
            pofs = par * PQB
            rofs = par * CR

            @pl.when(c + 1 < nchunk)
            def _prefetch():
                issue_in(c + 1, 1 - par)

            pltpu.make_async_copy(
                pq_hbm.at[pl.ds(0, PQB)],
                pq_v.at[pl.ds(0, PQB)],
                isem.at[par],
            ).wait()
            pltpu.make_async_copy(
                gtx_hbm.at[pl.ds(0, CR), :],
                gtx_v.at[pl.ds(0, CR), :],
                isem.at[par],
            ).wait()

            @pl.when(c >= 2)
            def _drain_out():
                pltpu.make_async_copy(
                    h_v.at[pl.ds(0, CR), :],
                    out_hbm.at[pl.ds(0, CR), :],
                    osem.at[par],
                ).wait()

            @plsc.parallel_loop(0, CR, step=4)
            def _rows(r0):
                for j in range(4):
                    rr = r0 + j
                    rbase = pofs + rr * (2 * F) + F
                    bv = [
                        gtx_v[rofs + rr, pl.ds(kk * 16, 16)] + pofs
                        for kk in range(K)
                    ]
                    for f0 in range(0, F, 16):
                        m = plsc.load_gather(pq_v, [bv[0] + f0])
                        for kk in range(1, K):
                            m = jnp.maximum(
                                m, plsc.load_gather(pq_v, [bv[kk] + f0])
                            )
                        h = m + pq_v[pl.ds(rbase + f0, 16)]
                        h = jnp.where(h >= 0, h, 0.2 * h)
                        h_v[rofs + rr, pl.ds(f0, 16)] = h
            pltpu.async_copy(
                h_v.at[pl.ds(rofs, CR), :],
                out_hbm.at[pl.ds(ch * CR, CR), :],
                osem.at[par],
            )
            return carry

        lax.fori_loop(0, nchunk, chunk_body, 0)
        for par in range(2):
            pltpu.make_async_copy(
                h_v.at[pl.ds(0, CR), :],
                out_hbm.at[pl.ds(0, CR), :],
                osem.at[par],
            ).wait()

    return run(pq_flat, gtx)


def _sc_assemble(h1a, h1b, h2a, h2b, h3a, h3b):
    mesh = plsc.VectorSubcoreMesh(
        core_axis_name="c", subcore_axis_name="s", num_cores=2, num_subcores=16
    )
    HROWS = R // 2

    @functools.partial(
        pl.kernel,
        out_type=jax.ShapeDtypeStruct((B * 3 * F, N), jnp.float32),
        mesh=mesh,
        compiler_params=pltpu.CompilerParams(needs_layout_passes=False),
        scratch_types=[
            pltpu.VMEM((CR, F), jnp.float32),
            pltpu.VMEM((CR, F + 1), jnp.float32),
            pltpu.VMEM((CB * 3 * F, N), jnp.float32),
        ],
    )
    def run(r1a, r1b, r2a, r2b, r3a, r3b, out_hbm, hst, hp, outc):
        wid = lax.axis_index("s") * 2 + lax.axis_index("c")
        lane = lax.iota(jnp.int32, 16)
        rvv = [g * 16 + lane for g in range(NGRP)]
        cb192 = [(rv // N) * (3 * F) for rv in rvv]
        nvv = [rv % N for rv in rvv]

        def chunk_body(c, carry):
            b0 = wid * BPW + c * CB
            rg = b0 * N
            rl = rg - (wid // (NW // 2)) * HROWS
            for l, (ra, rb) in enumerate(
                ((r1a, r1b), (r2a, r2b), (r3a, r3b))
            ):
                @pl.when(wid < NW // 2)
                def _stage_a():
                    pltpu.sync_copy(ra.at[pl.ds(rl, CR), :], hst)

                @pl.when(wid >= NW // 2)
                def _stage_b():
                    pltpu.sync_copy(rb.at[pl.ds(rl, CR), :], hst)

                @plsc.parallel_loop(0, CR, step=4)
                def _rp(r0):
                    for j in range(4):
                        rr = r0 + j
                        for f0 in range(0, F, 16):
                            hp[rr, pl.ds(f0, 16)] = hst[rr, pl.ds(f0, 16)]

                lF = l * F

                @plsc.parallel_loop(0, F, step=1)
                def _tp(f):
                    fsp = jnp.zeros((16,), jnp.int32) + f
                    for g in range(NGRP):
                        vec = plsc.load_gather(hp, [rvv[g], fsp])
                        plsc.store_scatter(
                            outc, [cb192[g] + (lF + f), nvv[g]], vec
                        )

            pltpu.sync_copy(
                outc, out_hbm.at[pl.ds(b0 * 3 * F, CB * 3 * F), :]
            )
            return carry

        lax.fori_loop(0, NCHUNK, chunk_body, 0)

    return run(h1a, h1b, h2a, h2b, h3a, h3b)


def _combine(W, b, cin):
    Wc = jnp.concatenate([W[:cin], W[cin:] - W[:cin]], axis=1)
    bc = jnp.concatenate([jnp.zeros((F,), jnp.float32), b]).reshape(1, 2 * F)
    return Wc, bc


def kernel(x, idx, W1, b1, W2, b2, W3, b3):
    xt = jnp.transpose(x, (0, 2, 1)).reshape(R, C)
    Wc1, bc1 = _combine(W1, b1, C)
    Wc2, bc2 = _combine(W2, b2, F)
    Wc3, bc3 = _combine(W3, b3, F)
    loc = (jnp.arange(B, dtype=jnp.int32) % CB)[:, None, None] * N + idx
    gtx = (
        loc.reshape(R, K)[:, :, None] * (2 * F)
        + jnp.arange(16, dtype=jnp.int32)
    ).reshape(R, K * 16)

    H = R // 2
    halves_h = []
    for xh, gh in ((xt[:H], gtx[:H]), (xt[H:], gtx[H:])):
        hs = []
        h = xh
        for Wc, bc in ((Wc1, bc1), (Wc2, bc2), (Wc3, bc3)):
            pq = _matmul(h, Wc, bc)
            h = _sc_gather_max(pq.reshape(-1), gh, rows=H)
            hs.append(h)
        halves_h.append(hs)

    out2 = _sc_assemble(
        halves_h[0][0], halves_h[1][0],
        halves_h[0][1], halves_h[1][1],
        halves_h[0][2], halves_h[1][2],
    )
    return out2.reshape(B, 3 * F, N)


# speedup vs baseline: 1.7117x; 1.5819x over previous
import functools

import jax
import jax.numpy as jnp
from jax import lax
from jax.experimental import pallas as pl
from jax.experimental.pallas import tpu as pltpu
from jax.experimental.pallas import tpu_sc as plsc

B, C, N, K = 1024, 128, 24, 4
R = B * N
F = 64
NW = 32
BPW = B // NW
CB = 4
CR = CB * N
NCHUNK = BPW // CB
NGRP = CR // 16


def _mm_body(x_ref, w_ref, b_ref, o_ref):
    o_ref[...] = (
        jnp.dot(x_ref[...], w_ref[...], preferred_element_type=jnp.float32)
        + b_ref[...]
    )


def _matmul(x, w, b, blk=2048):
    rows, cin = x.shape
    cout = w.shape[1]
    return pl.pallas_call(
        _mm_body,
        grid=(rows // blk,),
        in_specs=[
            pl.BlockSpec((blk, cin), lambda i: (i, 0)),
            pl.BlockSpec((cin, cout), lambda i: (0, 0)),
            pl.BlockSpec((1, cout), lambda i: (0, 0)),
        ],
        out_specs=pl.BlockSpec((blk, cout), lambda i: (i, 0)),
        out_shape=jax.ShapeDtypeStruct((rows, cout), jnp.float32),
    )(x, w, b)


def _sc_gather_max(pq_flat, gtx, rows=R):
    mesh = plsc.VectorSubcoreMesh(
        core_axis_name="c", subcore_axis_name="s", num_cores=2, num_subcores=16
    )

    nchunk = rows // (NW * CR)

    @functools.partial(
        pl.kernel,
        out_type=jax.ShapeDtypeStruct((rows, F), jnp.float32),
        mesh=mesh,
        compiler_params=pltpu.CompilerParams(needs_layout_passes=False),
        scratch_types=[
            pltpu.VMEM((2 * CR * 2 * F,), jnp.float32),
            pltpu.VMEM((2 * CR, K * 16), jnp.int32),
            pltpu.VMEM((2 * CR, F), jnp.float32),
            pltpu.SemaphoreType.DMA((2,)),
            pltpu.SemaphoreType.DMA((2,)),
        ],
    )
    def run(pq_hbm, gtx_hbm, out_hbm, pq_v, gtx_v, h_v, isem, osem):
        wid = lax.axis_index("s") * 2 + lax.axis_index("c")
        PQB = CR * 2 * F

        def issue_in(c, par):
            ch = wid * nchunk + c
            pltpu.async_copy(
                pq_hbm.at[pl.ds(ch * PQB, PQB)],
                pq_v.at[pl.ds(par * PQB, PQB)],
                isem.at[par],
            )
            pltpu.async_copy(
                gtx_hbm.at[pl.ds(ch * CR, CR), :],
                gtx_v.at[pl.ds(par * CR, CR), :],
                isem.at[par],
            )

        issue_in(0, 0)

        def chunk_body(c, carry):
            par = c % 2
            ch = wid * nchunk + c
            pofs = par * PQB
            rofs = par * CR

            @pl.when(c + 1 < nchunk)
            def _prefetch():
                issue_in(c + 1, 1 - par)

            pltpu.make_async_copy(
                pq_hbm.at[pl.ds(0, PQB)],
                pq_v.at[pl.ds(0, PQB)],
                isem.at[par],
            ).wait()
            pltpu.make_async_copy(
                gtx_hbm.at[pl.ds(0, CR), :],
                gtx_v.at[pl.ds(0, CR), :],
                isem.at[par],
            ).wait()

            @pl.when(c >= 2)
            def _drain_out():
                pltpu.make_async_copy(
                    h_v.at[pl.ds(0, CR), :],
                    out_hbm.at[pl.ds(0, CR), :],
                    osem.at[par],
                ).wait()

            @plsc.parallel_loop(0, CR, step=4)
            def _rows(r0):
                for j in range(4):
                    rr = r0 + j
                    rbase = pofs + rr * (2 * F) + F
                    bv = [
                        gtx_v[rofs + rr, pl.ds(kk * 16, 16)] + pofs
                        for kk in range(K)
                    ]
                    for f0 in range(0, F, 16):
                        m = plsc.load_gather(pq_v, [bv[0] + f0])
                        for kk in range(1, K):
                            m = jnp.maximum(
                                m, plsc.load_gather(pq_v, [bv[kk] + f0])
                            )
                        h = m + pq_v[pl.ds(rbase + f0, 16)]
                        h = jnp.where(h >= 0, h, 0.2 * h)
                        h_v[rofs + rr, pl.ds(f0, 16)] = h
            pltpu.async_copy(
                h_v.at[pl.ds(rofs, CR), :],
                out_hbm.at[pl.ds(ch * CR, CR), :],
                osem.at[par],
            )
            return carry

        lax.fori_loop(0, nchunk, chunk_body, 0)
        for par in range(2):
            pltpu.make_async_copy(
                h_v.at[pl.ds(0, CR), :],
                out_hbm.at[pl.ds(0, CR), :],
                osem.at[par],
            ).wait()

    return run(pq_flat, gtx)


def _combine(W, b, cin):
    Wc = jnp.concatenate([W[:cin], W[cin:] - W[:cin]], axis=1)
    bc = jnp.concatenate([jnp.zeros((F,), jnp.float32), b]).reshape(1, 2 * F)
    return Wc, bc


def kernel(x, idx, W1, b1, W2, b2, W3, b3):
    xt = jnp.transpose(x, (0, 2, 1)).reshape(R, C)
    Wc1, bc1 = _combine(W1, b1, C)
    Wc2, bc2 = _combine(W2, b2, F)
    Wc3, bc3 = _combine(W3, b3, F)
    loc = (jnp.arange(B, dtype=jnp.int32) % CB)[:, None, None] * N + idx
    gtx = (
        loc.reshape(R, K)[:, :, None] * (2 * F)
        + jnp.arange(16, dtype=jnp.int32)
    ).reshape(R, K * 16)

    H = R // 2
    halves_h = []
    for xh, gh in ((xt[:H], gtx[:H]), (xt[H:], gtx[H:])):
        hs = []
        h = xh
        for Wc, bc in ((Wc1, bc1), (Wc2, bc2), (Wc3, bc3)):
            pq = _matmul(h, Wc, bc)
            h = _sc_gather_max(pq.reshape(-1), gh, rows=H)
            hs.append(h)
        halves_h.append(hs)

    Bh = B // 2
    outs = []
    for hs in halves_h:
        oh = jnp.concatenate(hs, axis=1).reshape(Bh, N, 3 * F)
        outs.append(jnp.transpose(oh, (0, 2, 1)))
    return jnp.concatenate(outs, axis=0)


# speedup vs baseline: 1.7418x; 1.0176x over previous
import functools

import jax
import jax.numpy as jnp
from jax import lax
from jax.experimental import pallas as pl
from jax.experimental.pallas import tpu as pltpu
from jax.experimental.pallas import tpu_sc as plsc

B, C, N, K = 1024, 128, 24, 4
R = B * N
F = 64
NW = 32
BPW = B // NW
CB = 4
CR = CB * N
NCHUNK = BPW // CB
NGRP = CR // 16


def _mm_body(x_ref, w_ref, b_ref, o_ref):
    o_ref[...] = (
        jnp.dot(x_ref[...], w_ref[...], preferred_element_type=jnp.float32)
        + b_ref[...]
    )


def _matmul(x, w, b, blk=2048):
    rows, cin = x.shape
    cout = w.shape[1]
    return pl.pallas_call(
        _mm_body,
        grid=(rows // blk,),
        in_specs=[
            pl.BlockSpec((blk, cin), lambda i: (i, 0)),
            pl.BlockSpec((cin, cout), lambda i: (0, 0)),
            pl.BlockSpec((1, cout), lambda i: (0, 0)),
        ],
        out_specs=pl.BlockSpec((blk, cout), lambda i: (i, 0)),
        out_shape=jax.ShapeDtypeStruct((rows, cout), jnp.float32),
    )(x, w, b)


def _sc_gather_max(pq_flat, gtx, rows=R):
    mesh = plsc.VectorSubcoreMesh(
        core_axis_name="c", subcore_axis_name="s", num_cores=2, num_subcores=16
    )

    nchunk = rows // (NW * CR)

    @functools.partial(
        pl.kernel,
        out_type=jax.ShapeDtypeStruct((rows, F), jnp.float32),
        mesh=mesh,
        compiler_params=pltpu.CompilerParams(needs_layout_passes=False),
        scratch_types=[
            pltpu.VMEM((2 * CR * 2 * F,), jnp.float32),
            pltpu.VMEM((2 * CR, K * 16), jnp.int32),
            pltpu.VMEM((2 * CR, F), jnp.float32),
            pltpu.SemaphoreType.DMA((2,)),
            pltpu.SemaphoreType.DMA((2,)),
        ],
    )
    def run(pq_hbm, gtx_hbm, out_hbm, pq_v, gtx_v, h_v, isem, osem):
        wid = lax.axis_index("s") * 2 + lax.axis_index("c")
        PQB = CR * 2 * F

        def issue_in(c, par):
            ch = wid * nchunk + c
            pltpu.async_copy(
                pq_hbm.at[pl.ds(ch * PQB, PQB)],
                pq_v.at[pl.ds(par * PQB, PQB)],
                isem.at[par],
            )
            pltpu.async_copy(
                gtx_hbm.at[pl.ds(ch * CR, CR), :],
                gtx_v.at[pl.ds(par * CR, CR), :],
                isem.at[par],
            )

        issue_in(0, 0)

        def chunk_body(c, carry):
            par = c % 2
            ch = wid * nchunk + c
            pofs = par * PQB
            rofs = par * CR

            @pl.when(c + 1 < nchunk)
            def _prefetch():
                issue_in(c + 1, 1 - par)

            pltpu.make_async_copy(
                pq_hbm.at[pl.ds(0, PQB)],
                pq_v.at[pl.ds(0, PQB)],
                isem.at[par],
            ).wait()
            pltpu.make_async_copy(
                gtx_hbm.at[pl.ds(0, CR), :],
                gtx_v.at[pl.ds(0, CR), :],
                isem.at[par],
            ).wait()

            @pl.when(c >= 2)
            def _drain_out():
                pltpu.make_async_copy(
                    h_v.at[pl.ds(0, CR), :],
                    out_hbm.at[pl.ds(0, CR), :],
                    osem.at[par],
                ).wait()

            @plsc.parallel_loop(0, CR, step=4, unroll=2)
            def _rows(r0):
                for j in range(4):
                    rr = r0 + j
                    rbase = pofs + rr * (2 * F) + F
                    bv = [
                        gtx_v[rofs + rr, pl.ds(kk * 16, 16)] + pofs
                        for kk in range(K)
                    ]
                    for f0 in range(0, F, 16):
                        m = plsc.load_gather(pq_v, [bv[0] + f0])
                        for kk in range(1, K):
                            m = jnp.maximum(
                                m, plsc.load_gather(pq_v, [bv[kk] + f0])
                            )
                        h = m + pq_v[pl.ds(rbase + f0, 16)]
                        h = jnp.where(h >= 0, h, 0.2 * h)
                        h_v[rofs + rr, pl.ds(f0, 16)] = h
            pltpu.async_copy(
                h_v.at[pl.ds(rofs, CR), :],
                out_hbm.at[pl.ds(ch * CR, CR), :],
                osem.at[par],
            )
            return carry

        lax.fori_loop(0, nchunk, chunk_body, 0)
        for par in range(2):
            pltpu.make_async_copy(
                h_v.at[pl.ds(0, CR), :],
                out_hbm.at[pl.ds(0, CR), :],
                osem.at[par],
            ).wait()

    return run(pq_flat, gtx)


def _combine(W, b, cin):
    Wc = jnp.concatenate([W[:cin], W[cin:] - W[:cin]], axis=1)
    bc = jnp.concatenate([jnp.zeros((F,), jnp.float32), b]).reshape(1, 2 * F)
    return Wc, bc


def kernel(x, idx, W1, b1, W2, b2, W3, b3):
    xt = jnp.transpose(x, (0, 2, 1)).reshape(R, C)
    Wc1, bc1 = _combine(W1, b1, C)
    Wc2, bc2 = _combine(W2, b2, F)
    Wc3, bc3 = _combine(W3, b3, F)
    loc = (jnp.arange(B, dtype=jnp.int32) % CB)[:, None, None] * N + idx
    gtx = (
        loc.reshape(R, K)[:, :, None] * (2 * F)
        + jnp.arange(16, dtype=jnp.int32)
    ).reshape(R, K * 16)

    H = R // 2
    halves_h = []
    for xh, gh in ((xt[:H], gtx[:H]), (xt[H:], gtx[H:])):
        hs = []
        h = xh
        for Wc, bc in ((Wc1, bc1), (Wc2, bc2), (Wc3, bc3)):
            pq = _matmul(h, Wc, bc)
            h = _sc_gather_max(pq.reshape(-1), gh, rows=H)
            hs.append(h)
        halves_h.append(hs)

    h1 = jnp.concatenate([halves_h[0][0], halves_h[1][0]], axis=0)
    h2 = jnp.concatenate([halves_h[0][1], halves_h[1][1]], axis=0)
    h3 = jnp.concatenate([halves_h[0][2], halves_h[1][2]], axis=0)
    out = jnp.concatenate([h1, h2, h3], axis=1).reshape(B, N, 3 * F)
    return jnp.transpose(out, (0, 2, 1))


# speedup vs baseline: 1.7553x; 1.0077x over previous
import functools

import jax
import jax.numpy as jnp
from jax import lax
from jax.experimental import pallas as pl
from jax.experimental.pallas import tpu as pltpu
from jax.experimental.pallas import tpu_sc as plsc

B, C, N, K = 1024, 128, 24, 4
R = B * N
F = 64
NW = 32
BPW = B // NW
CB = 4
CR = CB * N
NCHUNK = BPW // CB
NGRP = CR // 16


def _mm_body(x_ref, w_ref, b_ref, o_ref):
    o_ref[...] = (
        jnp.dot(x_ref[...], w_ref[...], preferred_element_type=jnp.float32)
        + b_ref[...]
    )


def _matmul(x, w, b, blk=2048):
    rows, cin = x.shape
    cout = w.shape[1]
    return pl.pallas_call(
        _mm_body,
        grid=(rows // blk,),
        in_specs=[
            pl.BlockSpec((blk, cin), lambda i: (i, 0)),
            pl.BlockSpec((cin, cout), lambda i: (0, 0)),
            pl.BlockSpec((1, cout), lambda i: (0, 0)),
        ],
        out_specs=pl.BlockSpec((blk, cout), lambda i: (i, 0)),
        out_shape=jax.ShapeDtypeStruct((rows, cout), jnp.float32),
    )(x, w, b)


def _sc_gather_max(pq_flat, gtx, rows=R):
    mesh = plsc.VectorSubcoreMesh(
        core_axis_name="c", subcore_axis_name="s", num_cores=2, num_subcores=16
    )

    nchunk = rows // (NW * CR)

    @functools.partial(
        pl.kernel,
        out_type=jax.ShapeDtypeStruct((rows, F), jnp.float32),
        mesh=mesh,
        compiler_params=pltpu.CompilerParams(needs_layout_passes=False),
        scratch_types=[
            pltpu.VMEM((2 * CR * 2 * F,), jnp.float32),
            pltpu.VMEM((2 * CR, K * 16), jnp.int32),
            pltpu.VMEM((2 * CR, F), jnp.float32),
            pltpu.SemaphoreType.DMA((2,)),
            pltpu.SemaphoreType.DMA((2,)),
        ],
    )
    def run(pq_hbm, gtx_hbm, out_hbm, pq_v, gtx_v, h_v, isem, osem):
        wid = lax.axis_index("s") * 2 + lax.axis_index("c")
        PQB = CR * 2 * F

        def issue_in(c, par):
            ch = wid * nchunk + c
            pltpu.async_copy(
                pq_hbm.at[pl.ds(ch * PQB, PQB)],
                pq_v.at[pl.ds(par * PQB, PQB)],
                isem.at[par],
            )
            pltpu.async_copy(
                gtx_hbm.at[pl.ds(ch * CR, CR), :],
                gtx_v.at[pl.ds(par * CR, CR), :],
                isem.at[par],
            )

        issue_in(0, 0)

        def chunk_body(c, carry):
            par = c % 2
            ch = wid * nchunk + c
            pofs = par * PQB
            rofs = par * CR

            @pl.when(c + 1 < nchunk)
            def _prefetch():
                issue_in(c + 1, 1 - par)

            pltpu.make_async_copy(
                pq_hbm.at[pl.ds(0, PQB)],
                pq_v.at[pl.ds(0, PQB)],
                isem.at[par],
            ).wait()
            pltpu.make_async_copy(
                gtx_hbm.at[pl.ds(0, CR), :],
                gtx_v.at[pl.ds(0, CR), :],
                isem.at[par],
            ).wait()

            @pl.when(c >= 2)
            def _drain_out():
                pltpu.make_async_copy(
                    h_v.at[pl.ds(0, CR), :],
                    out_hbm.at[pl.ds(0, CR), :],
                    osem.at[par],
                ).wait()

            @plsc.parallel_loop(0, CR, step=4)
            def _rows(r0):
                for j in range(4):
                    rr = r0 + j
                    rbase = pofs + rr * (2 * F) + F
                    bv = [
                        gtx_v[rofs + rr, pl.ds(kk * 16, 16)] + pofs
                        for kk in range(K)
                    ]
                    for f0 in range(0, F, 16):
                        m = plsc.load_gather(pq_v, [bv[0] + f0])
                        for kk in range(1, K):
                            m = jnp.maximum(
                                m, plsc.load_gather(pq_v, [bv[kk] + f0])
                            )
                        h = m + pq_v[pl.ds(rbase + f0, 16)]
                        h = jnp.where(h >= 0, h, 0.2 * h)
                        h_v[rofs + rr, pl.ds(f0, 16)] = h
            pltpu.async_copy(
                h_v.at[pl.ds(rofs, CR), :],
                out_hbm.at[pl.ds(ch * CR, CR), :],
                osem.at[par],
            )
            return carry

        lax.fori_loop(0, nchunk, chunk_body, 0)
        for par in range(2):
            pltpu.make_async_copy(
                h_v.at[pl.ds(0, CR), :],
                out_hbm.at[pl.ds(0, CR), :],
                osem.at[par],
            ).wait()

    return run(pq_flat, gtx)


def _combine(W, b, cin):
    Wc = jnp.concatenate([W[:cin], W[cin:] - W[:cin]], axis=1)
    bc = jnp.concatenate([jnp.zeros((F,), jnp.float32), b]).reshape(1, 2 * F)
    return Wc, bc


def kernel(x, idx, W1, b1, W2, b2, W3, b3):
    xt = jnp.transpose(x, (0, 2, 1)).reshape(R, C)
    Wc1, bc1 = _combine(W1, b1, C)
    Wc2, bc2 = _combine(W2, b2, F)
    Wc3, bc3 = _combine(W3, b3, F)
    loc = (jnp.arange(B, dtype=jnp.int32) % CB)[:, None, None] * N + idx
    gtx = (
        loc.reshape(R, K)[:, :, None] * (2 * F)
        + jnp.arange(16, dtype=jnp.int32)
    ).reshape(R, K * 16)

    H = R // 2
    halves_h = []
    for xh, gh in ((xt[:H], gtx[:H]), (xt[H:], gtx[H:])):
        hs = []
        h = xh
        for Wc, bc in ((Wc1, bc1), (Wc2, bc2), (Wc3, bc3)):
            pq = _matmul(h, Wc, bc)
            h = _sc_gather_max(pq.reshape(-1), gh, rows=H)
            hs.append(h)
        halves_h.append(hs)

    h1 = jnp.concatenate([halves_h[0][0], halves_h[1][0]], axis=0)
    h2 = jnp.concatenate([halves_h[0][1], halves_h[1][1]], axis=0)
    h3 = jnp.concatenate([halves_h[0][2], halves_h[1][2]], axis=0)
    out = jnp.concatenate([h1, h2, h3], axis=1).reshape(B, N, 3 * F)
    return jnp.transpose(out, (0, 2, 1))
